# baseline (speedup 1.0000x reference)
import functools

import numpy as np
import jax
import jax.numpy as jnp
from jax import lax
from jax.experimental import pallas as pl
from jax.experimental.pallas import tpu as pltpu
from jax.experimental.pallas import tpu_sc as plsc

N = 100000
E = 3200000
S_IN = 128
V_IN = 16
S_OUT = 128
V_OUT = 16
HID = 16
EPS = 1e-8

_NW = 32
_SB = 128
_KB = 16
_CHUNK = _SB * _KB
_NCH = 49
_NCH_LAST = 43
_KB_TAIL = 8
_RPS = 6256
_N_PAD = 16 * _RPS
_PSTR = _CHUNK + 8

_SC_PARAMS = pltpu.CompilerParams(use_tc_tiling_on_sc=False)
_MESH = dict(core_axis_name="c", subcore_axis_name="s")


def _worker_loop(w, do_chunk):
    def chunk_body(c, carry):
        do_chunk(w * _NCH + c, _KB)
        return carry

    nch = jnp.where(w < _NW - 1, _NCH, _NCH_LAST)
    lax.fori_loop(0, nch, chunk_body, 0)

    @pl.when(w == _NW - 1)
    def _tail():
        do_chunk((_NW - 1) * _NCH + _NCH_LAST, _KB_TAIL)


_PP = np.zeros((128, 128), np.float32)
for _c in range(8):
    for _k in range(16):
        _PP[_c * 16 + _k, _c + 8 * _k] = 1.0

_RR = 200
_ROWS_P = E // 128
_STEPS = _ROWS_P // _RR


def _tc_repack_body(*refs):
    plane_refs = refs[:8]
    pp_ref, out_ref = refs[8], refs[9]
    pp = pp_ref[...]
    accs = []
    for w in range(8):
        xw = jnp.concatenate(
            [plane_refs[c][:, 16 * w:16 * w + 16] for c in range(8)], axis=1)
        accs.append(jnp.dot(xw, pp, preferred_element_type=jnp.float32))
    stacked = jnp.stack(accs, axis=1)
    out_ref[...] = stacked.reshape(_RR * 8, 128)


def _tc_repack(fr2d, pp):
    def plane_spec(c):
        return pl.BlockSpec((_RR, 128), lambda i, c=c: (c * _STEPS + i, 0))

    return pl.pallas_call(
        _tc_repack_body,
        grid=(_STEPS,),
        in_specs=[plane_spec(c) for c in range(8)]
        + [pl.BlockSpec((128, 128), lambda i: (0, 0))],
        out_specs=pl.BlockSpec((_RR * 8, 128), lambda i: (i, 0)),
        out_shape=jax.ShapeDtypeStruct((E // 16, 128), jnp.float32),
    )(*([fr2d] * 8 + [pp]))


def _sc_segment_sum(row2d, val8_2d, frames_flat, zeros8, zeros1):
    mesh = plsc.VectorSubcoreMesh(**_MESH)

    @functools.partial(
        pl.kernel,
        out_type=[
            jax.ShapeDtypeStruct((2, _N_PAD, 8), jnp.float32),
            jax.ShapeDtypeStruct((2, _N_PAD), jnp.float32),
            jax.ShapeDtypeStruct((2, _N_PAD), jnp.float32),
        ],
        mesh=mesh,
        scratch_types=[
            pltpu.VMEM((_KB, _SB), jnp.int32),
            pltpu.VMEM((_CHUNK, 8), jnp.float32),
            pltpu.VMEM((_CHUNK,), jnp.float32),
            pltpu.VMEM((_SB,), jnp.float32),
            pltpu.VMEM_SHARED((_N_PAD, 8), jnp.float32),
            pltpu.VMEM_SHARED((_N_PAD,), jnp.float32),
            pltpu.VMEM_SHARED((_N_PAD,), jnp.float32),
        ],
        compiler_params=_SC_PARAMS,
    )
    def k(row_hbm, val8_hbm, frames_hbm, zeros8_hbm, zeros1_hbm,
          out8, outc8, outcnt, idx_v, val8_v, col8_v, ones_v,
          t8, tc8, tcnt):
        cid = lax.axis_index("c")
        sid = lax.axis_index("s")
        w = cid * 16 + sid

        r0 = sid * _RPS
        pltpu.sync_copy(zeros8_hbm.at[pl.ds(r0, _RPS)], t8.at[pl.ds(r0, _RPS)])
        pltpu.sync_copy(zeros1_hbm.at[pl.ds(r0, _RPS)], tc8.at[pl.ds(r0, _RPS)])
        pltpu.sync_copy(zeros1_hbm.at[pl.ds(r0, _RPS)], tcnt.at[pl.ds(r0, _RPS)])

        def fill(i, carry):
            ones_v[pl.ds(i * 16, 16)] = jnp.full((16,), 1.0, jnp.float32)
            return carry

        lax.fori_loop(0, _SB // 16, fill, 0)
        plsc.subcore_barrier()

        def do_chunk(base, kb):
            n = kb * _SB
            e0 = base * _CHUNK
            pltpu.sync_copy(row_hbm.at[pl.ds(base * _KB, kb)],
                            idx_v.at[pl.ds(0, kb)])
            pltpu.sync_copy(val8_hbm.at[pl.ds(e0, n)],
                            val8_v.at[pl.ds(0, n)])
            pltpu.sync_copy(frames_hbm.at[pl.ds(8 * E + e0, n)],
                            col8_v.at[pl.ds(0, n)])
            for j in range(kb):
                pltpu.sync_copy(val8_v.at[pl.ds(j * _SB, _SB)],
                                t8.at[idx_v.at[j]], add=True)
                pltpu.sync_copy(col8_v.at[pl.ds(j * _SB, _SB)],
                                tc8.at[idx_v.at[j]], add=True)
                pltpu.sync_copy(ones_v, tcnt.at[idx_v.at[j]], add=True)

        _worker_loop(w, do_chunk)
        plsc.subcore_barrier()

        pltpu.sync_copy(t8.at[pl.ds(r0, _RPS)], out8.at[cid, pl.ds(r0, _RPS)])
        pltpu.sync_copy(tc8.at[pl.ds(r0, _RPS)], outc8.at[cid, pl.ds(r0, _RPS)])
        pltpu.sync_copy(tcnt.at[pl.ds(r0, _RPS)], outcnt.at[cid, pl.ds(r0, _RPS)])

    return k(row2d, val8_2d, frames_flat, zeros8, zeros1)


_A8 = np.zeros((8, 27), np.float32)
_A1 = np.zeros((1, 27), np.float32)
_B9 = np.zeros((9, 27), np.float32)
_C27 = np.zeros((27, 9), np.float32)
for _i in range(3):
    for _j in range(3):
        for _c in range(3):
            _e = _i * 9 + _j * 3 + _c
            _a = _j * 3 + _c
            if _a < 8:
                _A8[_a, _e] = 1.0
            else:
                _A1[0, _e] = 1.0
            _B9[_i * 3 + _c, _e] = 1.0
            _C27[_e, _i * 3 + _j] = 1.0

_BLK = 1000


def _tc_body(sc_ref, x_ref, t80_ref, t81_ref, c80_ref, c81_ref, cn0_ref,
             cn1_ref, wvd_ref, g16_ref, wvdf_ref, a8_ref, a1_ref, sos_ref,
             sov_ref, sou_ref, bso_ref, wvos_ref, bvos_ref, wvu_ref, k48_ref,
             sout_ref, vout_ref):
    x = x_ref[...]
    vh = jnp.dot(x, wvd_ref[...], preferred_element_type=jnp.float32)
    vnsq = jnp.dot(vh * vh, g16_ref[...], preferred_element_type=jnp.float32)
    vn = jnp.sqrt(vnsq + EPS)
    vdf27 = jnp.dot(x, wvdf_ref[...], preferred_element_type=jnp.float32)
    t8 = t80_ref[...] + t81_ref[...]
    c8 = c80_ref[...] + c81_ref[...]
    cnt = cn0_ref[...] + cn1_ref[...]
    inv = 1.0 / jnp.maximum(cnt, 1.0)
    mean27 = (jnp.dot(t8, a8_ref[...], preferred_element_type=jnp.float32)
              + jnp.dot(c8, a1_ref[...], preferred_element_type=jnp.float32)
              ) * inv
    u = mean27 * vdf27
    s = (jnp.dot(sc_ref[...], sos_ref[...], preferred_element_type=jnp.float32)
         + jnp.dot(vn, sov_ref[...], preferred_element_type=jnp.float32)
         + jnp.dot(u, sou_ref[...], preferred_element_type=jnp.float32)
         + bso_ref[...])
    sil = s * jax.nn.sigmoid(s)
    gate = jnp.dot(sil, wvos_ref[...],
                   preferred_element_type=jnp.float32) + bvos_ref[...]
    g48 = jnp.dot(jax.nn.sigmoid(gate), k48_ref[...],
                  preferred_element_type=jnp.float32)
    vout = jnp.dot(vh, wvu_ref[...], preferred_element_type=jnp.float32) * g48
    sout_ref[...] = sil
    vout_ref[...] = vout


def _tc_node(scalar_rep, x48, t80, t81, c80, c81, cn0, cn1, wvd48, g16,
             wvdf27, a8, a1, sos, sov, sou, bso, wvos_t, bvos, wvu48, k48):
    grid = (N // _BLK,)

    def blk(shape):
        return pl.BlockSpec((_BLK,) + shape[1:], lambda i: (i,) + (0,) * (len(shape) - 1))

    def full(shape):
        return pl.BlockSpec(shape, lambda i: (0,) * len(shape))

    return pl.pallas_call(
        _tc_body,
        grid=grid,
        in_specs=[
            blk((N, S_IN)), blk((N, 48)),
            blk((_N_PAD, 8)), blk((_N_PAD, 8)),
            blk((_N_PAD, 1)), blk((_N_PAD, 1)),
            blk((_N_PAD, 1)), blk((_N_PAD, 1)),
            full((48, 48)), full((48, 16)), full((48, 27)),
            full((8, 27)), full((1, 27)),
            full((S_IN, S_OUT)), full((16, S_OUT)), full((27, S_OUT)),
            full((1, S_OUT)), full((S_OUT, V_OUT)), full((1, V_OUT)),
            full((48, 48)), full((16, 48)),
        ],
        out_specs=[blk((N, S_OUT)), blk((N, 48))],
        out_shape=[
            jax.ShapeDtypeStruct((N, S_OUT), jnp.float32),
            jax.ShapeDtypeStruct((N, 48), jnp.float32),
        ],
    )(scalar_rep, x48, t80, t81, c80, c81, cn0, cn1, wvd48, g16, wvdf27,
      a8, a1, sos, sov, sou, bso, wvos_t, bvos, wvu48, k48)


def kernel(scalar_rep, vector_rep, edge_index, frames,
           W_vd, W_vdf, W_so, b_so, W_vu, W_vos, b_vos):
    row2d = edge_index[0].reshape(E // _SB, _SB)
    frames_flat = jnp.transpose(frames, (1, 2, 0)).reshape(-1)
    val8_2d = _tc_repack(frames_flat.reshape(9 * E // 128, 128),
                         jnp.asarray(_PP)).reshape(E, 8)
    zeros8 = jnp.zeros((_N_PAD, 8), jnp.float32)
    zeros1 = jnp.zeros((_N_PAD,), jnp.float32)
    out8, outc8, outcnt = _sc_segment_sum(
        row2d, val8_2d, frames_flat, zeros8, zeros1)

    i3 = jnp.eye(3, dtype=jnp.float32)
    i16 = jnp.eye(16, dtype=jnp.float32)
    wvd48 = jnp.kron(W_vd.T, i3)
    g16 = jnp.kron(i16, jnp.ones((3, 1), jnp.float32))
    wvdf27 = jnp.kron(W_vdf.T, i3) @ jnp.asarray(_B9)
    so_t = W_so.T
    sos = so_t[:S_IN]
    sov = so_t[S_IN:S_IN + HID]
    sou = jnp.asarray(_C27) @ so_t[S_IN + HID:]
    bso = b_so[None, :]
    wvos_t = W_vos.T
    bvos = b_vos[None, :]
    wvu48 = jnp.kron(W_vu.T, i3)
    k48 = jnp.kron(i16, jnp.ones((1, 3), jnp.float32))

    sout, vout48 = _tc_node(
        scalar_rep, vector_rep.reshape(N, 48),
        out8[0], out8[1],
        outc8[0].reshape(_N_PAD, 1), outc8[1].reshape(_N_PAD, 1),
        outcnt[0].reshape(_N_PAD, 1), outcnt[1].reshape(_N_PAD, 1),
        wvd48, g16, wvdf27, jnp.asarray(_A8), jnp.asarray(_A1),
        sos, sov, sou, bso, wvos_t, bvos, wvu48, k48)
    return sout, vout48.reshape(N, V_OUT, 3)


# speedup vs baseline: 4.5167x; 4.5167x over previous
import functools

import numpy as np
import jax
import jax.numpy as jnp
from jax import lax
from jax.experimental import pallas as pl
from jax.experimental.pallas import tpu as pltpu
from jax.experimental.pallas import tpu_sc as plsc

N = 100000
E = 3200000
S_IN = 128
V_IN = 16
S_OUT = 128
V_OUT = 16
HID = 16
EPS = 1e-8

_NW = 32
_SB = 128
_KB = 16
_CHUNK = _SB * _KB
_NCH = 49
_NCH_LAST = 43
_KB_TAIL = 8
_RPS = 6256
_N_PAD = 16 * _RPS
_PSTR = _CHUNK + 8

_SC_PARAMS = pltpu.CompilerParams(use_tc_tiling_on_sc=False)
_MESH = dict(core_axis_name="c", subcore_axis_name="s")


def _worker_loop(w, do_chunk):
    def chunk_body(c, carry):
        do_chunk(w * _NCH + c, _KB)
        return carry

    nch = jnp.where(w < _NW - 1, _NCH, _NCH_LAST)
    lax.fori_loop(0, nch, chunk_body, 0)

    @pl.when(w == _NW - 1)
    def _tail():
        do_chunk((_NW - 1) * _NCH + _NCH_LAST, _KB_TAIL)


_PP = np.zeros((128, 128), np.float32)
for _c in range(8):
    for _k in range(16):
        _PP[_c * 16 + _k, _c + 8 * _k] = 1.0

_RR = 200
_ROWS_P = E // 128
_STEPS = _ROWS_P // _RR


def _tc_repack_body(*refs):
    plane_refs = refs[:8]
    pp_ref, out_ref = refs[8], refs[9]
    pp = pp_ref[...]
    accs = []
    for w in range(8):
        xw = jnp.concatenate(
            [plane_refs[c][:, 16 * w:16 * w + 16] for c in range(8)], axis=1)
        accs.append(jnp.dot(xw, pp, preferred_element_type=jnp.float32))
    stacked = jnp.stack(accs, axis=1)
    out_ref[...] = stacked.reshape(_RR * 8, 128)


def _tc_repack(planes8, pp):
    return pl.pallas_call(
        _tc_repack_body,
        grid=(_STEPS,),
        in_specs=[pl.BlockSpec((_RR, 128), lambda i: (i, 0))
                  for _ in range(8)]
        + [pl.BlockSpec((128, 128), lambda i: (0, 0))],
        out_specs=pl.BlockSpec((_RR * 8, 128), lambda i: (i, 0)),
        out_shape=jax.ShapeDtypeStruct((E // 16, 128), jnp.float32),
    )(*(list(planes8) + [pp]))


def _sc_segment_sum(row2d, val8_2d, plane8, zeros8, zeros1):
    mesh = plsc.VectorSubcoreMesh(**_MESH)

    @functools.partial(
        pl.kernel,
        out_type=[
            jax.ShapeDtypeStruct((2, _N_PAD, 8), jnp.float32),
            jax.ShapeDtypeStruct((2, _N_PAD), jnp.float32),
            jax.ShapeDtypeStruct((2, _N_PAD), jnp.float32),
        ],
        mesh=mesh,
        scratch_types=[
            pltpu.VMEM((_KB, _SB), jnp.int32),
            pltpu.VMEM((_CHUNK, 8), jnp.float32),
            pltpu.VMEM((_CHUNK,), jnp.float32),
            pltpu.VMEM((_SB,), jnp.float32),
            pltpu.VMEM_SHARED((_N_PAD, 8), jnp.float32),
            pltpu.VMEM_SHARED((_N_PAD,), jnp.float32),
            pltpu.VMEM_SHARED((_N_PAD,), jnp.float32),
        ],
        compiler_params=_SC_PARAMS,
    )
    def k(row_hbm, val8_hbm, col8_hbm, zeros8_hbm, zeros1_hbm,
          out8, outc8, outcnt, idx_v, val8_v, col8_v, ones_v,
          t8, tc8, tcnt):
        cid = lax.axis_index("c")
        sid = lax.axis_index("s")
        w = cid * 16 + sid

        r0 = sid * _RPS
        pltpu.sync_copy(zeros8_hbm.at[pl.ds(r0, _RPS)], t8.at[pl.ds(r0, _RPS)])
        pltpu.sync_copy(zeros1_hbm.at[pl.ds(r0, _RPS)], tc8.at[pl.ds(r0, _RPS)])
        pltpu.sync_copy(zeros1_hbm.at[pl.ds(r0, _RPS)], tcnt.at[pl.ds(r0, _RPS)])

        def fill(i, carry):
            ones_v[pl.ds(i * 16, 16)] = jnp.full((16,), 1.0, jnp.float32)
            return carry

        lax.fori_loop(0, _SB // 16, fill, 0)
        plsc.subcore_barrier()

        def do_chunk(base, kb):
            n = kb * _SB
            e0 = base * _CHUNK
            pltpu.sync_copy(row_hbm.at[pl.ds(base * _KB, kb)],
                            idx_v.at[pl.ds(0, kb)])
            pltpu.sync_copy(val8_hbm.at[pl.ds(e0, n)],
                            val8_v.at[pl.ds(0, n)])
            pltpu.sync_copy(col8_hbm.at[pl.ds(e0, n)],
                            col8_v.at[pl.ds(0, n)])
            for j in range(kb):
                pltpu.sync_copy(val8_v.at[pl.ds(j * _SB, _SB)],
                                t8.at[idx_v.at[j]], add=True)
                pltpu.sync_copy(col8_v.at[pl.ds(j * _SB, _SB)],
                                tc8.at[idx_v.at[j]], add=True)
                pltpu.sync_copy(ones_v, tcnt.at[idx_v.at[j]], add=True)

        _worker_loop(w, do_chunk)
        plsc.subcore_barrier()

        pltpu.sync_copy(t8.at[pl.ds(r0, _RPS)], out8.at[cid, pl.ds(r0, _RPS)])
        pltpu.sync_copy(tc8.at[pl.ds(r0, _RPS)], outc8.at[cid, pl.ds(r0, _RPS)])
        pltpu.sync_copy(tcnt.at[pl.ds(r0, _RPS)], outcnt.at[cid, pl.ds(r0, _RPS)])

    return k(row2d, val8_2d, plane8, zeros8, zeros1)


_A8 = np.zeros((8, 27), np.float32)
_A1 = np.zeros((1, 27), np.float32)
_B9 = np.zeros((9, 27), np.float32)
_C27 = np.zeros((27, 9), np.float32)
for _i in range(3):
    for _j in range(3):
        for _c in range(3):
            _e = _i * 9 + _j * 3 + _c
            _a = _j * 3 + _c
            if _a < 8:
                _A8[_a, _e] = 1.0
            else:
                _A1[0, _e] = 1.0
            _B9[_i * 3 + _c, _e] = 1.0
            _C27[_e, _i * 3 + _j] = 1.0

_BLK = 1000


def _tc_body(sc_ref, x_ref, t80_ref, t81_ref, c80_ref, c81_ref, cn0_ref,
             cn1_ref, wvd_ref, g16_ref, wvdf_ref, a8_ref, a1_ref, sos_ref,
             sov_ref, sou_ref, bso_ref, wvos_ref, bvos_ref, wvu_ref, k48_ref,
             sout_ref, vout_ref):
    x = x_ref[...]
    vh = jnp.dot(x, wvd_ref[...], preferred_element_type=jnp.float32)
    vnsq = jnp.dot(vh * vh, g16_ref[...], preferred_element_type=jnp.float32)
    vn = jnp.sqrt(vnsq + EPS)
    vdf27 = jnp.dot(x, wvdf_ref[...], preferred_element_type=jnp.float32)
    t8 = t80_ref[...] + t81_ref[...]
    c8 = c80_ref[...] + c81_ref[...]
    cnt = cn0_ref[...] + cn1_ref[...]
    inv = 1.0 / jnp.maximum(cnt, 1.0)
    mean27 = (jnp.dot(t8, a8_ref[...], preferred_element_type=jnp.float32)
              + jnp.dot(c8, a1_ref[...], preferred_element_type=jnp.float32)
              ) * inv
    u = mean27 * vdf27
    s = (jnp.dot(sc_ref[...], sos_ref[...], preferred_element_type=jnp.float32)
         + jnp.dot(vn, sov_ref[...], preferred_element_type=jnp.float32)
         + jnp.dot(u, sou_ref[...], preferred_element_type=jnp.float32)
         + bso_ref[...])
    sil = s * jax.nn.sigmoid(s)
    gate = jnp.dot(sil, wvos_ref[...],
                   preferred_element_type=jnp.float32) + bvos_ref[...]
    g48 = jnp.dot(jax.nn.sigmoid(gate), k48_ref[...],
                  preferred_element_type=jnp.float32)
    vout = jnp.dot(vh, wvu_ref[...], preferred_element_type=jnp.float32) * g48
    sout_ref[...] = sil
    vout_ref[...] = vout


def _tc_node(scalar_rep, x48, t80, t81, c80, c81, cn0, cn1, wvd48, g16,
             wvdf27, a8, a1, sos, sov, sou, bso, wvos_t, bvos, wvu48, k48):
    grid = (N // _BLK,)

    def blk(shape):
        return pl.BlockSpec((_BLK,) + shape[1:], lambda i: (i,) + (0,) * (len(shape) - 1))

    def full(shape):
        return pl.BlockSpec(shape, lambda i: (0,) * len(shape))

    return pl.pallas_call(
        _tc_body,
        grid=grid,
        in_specs=[
            blk((N, S_IN)), blk((N, 48)),
            blk((_N_PAD, 8)), blk((_N_PAD, 8)),
            blk((_N_PAD, 1)), blk((_N_PAD, 1)),
            blk((_N_PAD, 1)), blk((_N_PAD, 1)),
            full((48, 48)), full((48, 16)), full((48, 27)),
            full((8, 27)), full((1, 27)),
            full((S_IN, S_OUT)), full((16, S_OUT)), full((27, S_OUT)),
            full((1, S_OUT)), full((S_OUT, V_OUT)), full((1, V_OUT)),
            full((48, 48)), full((16, 48)),
        ],
        out_specs=[blk((N, S_OUT)), blk((N, 48))],
        out_shape=[
            jax.ShapeDtypeStruct((N, S_OUT), jnp.float32),
            jax.ShapeDtypeStruct((N, 48), jnp.float32),
        ],
    )(scalar_rep, x48, t80, t81, c80, c81, cn0, cn1, wvd48, g16, wvdf27,
      a8, a1, sos, sov, sou, bso, wvos_t, bvos, wvu48, k48)


def kernel(scalar_rep, vector_rep, edge_index, frames,
           W_vd, W_vdf, W_so, b_so, W_vu, W_vos, b_vos):
    row2d = edge_index[0].reshape(E // _SB, _SB)
    planes = [frames[:, j, k] for j in range(3) for k in range(3)]
    val8_2d = _tc_repack([p.reshape(E // 128, 128) for p in planes[:8]],
                         jnp.asarray(_PP)).reshape(E, 8)
    zeros8 = jnp.zeros((_N_PAD, 8), jnp.float32)
    zeros1 = jnp.zeros((_N_PAD,), jnp.float32)
    out8, outc8, outcnt = _sc_segment_sum(
        row2d, val8_2d, planes[8], zeros8, zeros1)

    i3 = jnp.eye(3, dtype=jnp.float32)
    i16 = jnp.eye(16, dtype=jnp.float32)
    wvd48 = jnp.kron(W_vd.T, i3)
    g16 = jnp.kron(i16, jnp.ones((3, 1), jnp.float32))
    wvdf27 = jnp.kron(W_vdf.T, i3) @ jnp.asarray(_B9)
    so_t = W_so.T
    sos = so_t[:S_IN]
    sov = so_t[S_IN:S_IN + HID]
    sou = jnp.asarray(_C27) @ so_t[S_IN + HID:]
    bso = b_so[None, :]
    wvos_t = W_vos.T
    bvos = b_vos[None, :]
    wvu48 = jnp.kron(W_vu.T, i3)
    k48 = jnp.kron(i16, jnp.ones((1, 3), jnp.float32))

    sout, vout48 = _tc_node(
        scalar_rep, vector_rep.reshape(N, 48),
        out8[0], out8[1],
        outc8[0].reshape(_N_PAD, 1), outc8[1].reshape(_N_PAD, 1),
        outcnt[0].reshape(_N_PAD, 1), outcnt[1].reshape(_N_PAD, 1),
        wvd48, g16, wvdf27, jnp.asarray(_A8), jnp.asarray(_A1),
        sos, sov, sou, bso, wvos_t, bvos, wvu48, k48)
    return sout, vout48.reshape(N, V_OUT, 3)


# speedup vs baseline: 5.3237x; 1.1787x over previous
import functools

import numpy as np
import jax
import jax.numpy as jnp
from jax import lax
from jax.experimental import pallas as pl
from jax.experimental.pallas import tpu as pltpu
from jax.experimental.pallas import tpu_sc as plsc

N = 100000
E = 3200000
S_IN = 128
V_IN = 16
S_OUT = 128
V_OUT = 16
HID = 16
EPS = 1e-8

_NW = 32
_SB = 128
_KB = 16
_CHUNK = _SB * _KB
_NCH = 49
_NCH_LAST = 43
_KB_TAIL = 8
_RPS = 6256
_N_PAD = 16 * _RPS
_PSTR = _CHUNK + 8

_SC_PARAMS = pltpu.CompilerParams(use_tc_tiling_on_sc=False)
_MESH = dict(core_axis_name="c", subcore_axis_name="s")


def _worker_loop(w, do_chunk):
    def chunk_body(c, carry):
        do_chunk(w * _NCH + c, _KB)
        return carry

    nch = jnp.where(w < _NW - 1, _NCH, _NCH_LAST)
    lax.fori_loop(0, nch, chunk_body, 0)

    @pl.when(w == _NW - 1)
    def _tail():
        do_chunk((_NW - 1) * _NCH + _NCH_LAST, _KB_TAIL)


_PP = np.zeros((128, 128), np.float32)
for _c in range(8):
    for _k in range(16):
        _PP[_c * 16 + _k, _c + 8 * _k] = 1.0

_RR = 200
_ROWS_P = E // 128
_STEPS = _ROWS_P // _RR


def _tc_repack_body(*refs):
    plane_refs = refs[:8]
    pp_ref, out_ref = refs[8], refs[9]
    pp = pp_ref[...]
    accs = []
    for w in range(8):
        xw = jnp.concatenate(
            [plane_refs[c][:, 16 * w:16 * w + 16] for c in range(8)], axis=1)
        accs.append(jnp.dot(xw, pp, preferred_element_type=jnp.float32))
    stacked = jnp.stack(accs, axis=1)
    out_ref[...] = stacked.reshape(_RR * 8, 128)


def _tc_repack(planes8, pp):
    return pl.pallas_call(
        _tc_repack_body,
        grid=(_STEPS,),
        in_specs=[pl.BlockSpec((_RR, 128), lambda i: (i, 0))
                  for _ in range(8)]
        + [pl.BlockSpec((128, 128), lambda i: (0, 0))],
        out_specs=pl.BlockSpec((_RR * 8, 128), lambda i: (i, 0)),
        out_shape=jax.ShapeDtypeStruct((E // 16, 128), jnp.float32),
    )(*(list(planes8) + [pp]))


def _sc_segment_sum(row2d, val8_2d, plane8, zeros8, zeros1):
    mesh = plsc.VectorSubcoreMesh(**_MESH)

    @functools.partial(
        pl.kernel,
        out_type=[
            jax.ShapeDtypeStruct((2, _N_PAD, 8), jnp.float32),
            jax.ShapeDtypeStruct((2, _N_PAD), jnp.float32),
            jax.ShapeDtypeStruct((2, _N_PAD), jnp.float32),
        ],
        mesh=mesh,
        scratch_types=[
            pltpu.VMEM((_KB, _SB), jnp.int32),
            pltpu.VMEM((_CHUNK, 8), jnp.float32),
            pltpu.VMEM((_CHUNK,), jnp.float32),
            pltpu.VMEM((_SB,), jnp.float32),
            pltpu.VMEM_SHARED((_N_PAD, 8), jnp.float32),
            pltpu.VMEM_SHARED((_N_PAD,), jnp.float32),
            pltpu.VMEM_SHARED((_N_PAD,), jnp.float32),
            pltpu.SemaphoreType.DMA,
            pltpu.SemaphoreType.DMA,
        ],
        compiler_params=_SC_PARAMS,
    )
    def k(row_hbm, val8_hbm, col8_hbm, zeros8_hbm, zeros1_hbm,
          out8, outc8, outcnt, idx_v, val8_v, col8_v, ones_v,
          t8, tc8, tcnt, sem_in, sem_sc):
        cid = lax.axis_index("c")
        sid = lax.axis_index("s")
        w = cid * 16 + sid

        r0 = sid * _RPS
        pltpu.sync_copy(zeros8_hbm.at[pl.ds(r0, _RPS)], t8.at[pl.ds(r0, _RPS)])
        pltpu.sync_copy(zeros1_hbm.at[pl.ds(r0, _RPS)], tc8.at[pl.ds(r0, _RPS)])
        pltpu.sync_copy(zeros1_hbm.at[pl.ds(r0, _RPS)], tcnt.at[pl.ds(r0, _RPS)])

        def fill(i, carry):
            ones_v[pl.ds(i * 16, 16)] = jnp.full((16,), 1.0, jnp.float32)
            return carry

        lax.fori_loop(0, _SB // 16, fill, 0)
        plsc.subcore_barrier()

        def do_chunk(base, kb):
            n = kb * _SB
            e0 = base * _CHUNK
            loads = [
                pltpu.async_copy(row_hbm.at[pl.ds(base * _KB, kb)],
                                 idx_v.at[pl.ds(0, kb)], sem_in),
                pltpu.async_copy(val8_hbm.at[pl.ds(e0, n)],
                                 val8_v.at[pl.ds(0, n)], sem_in),
                pltpu.async_copy(col8_hbm.at[pl.ds(e0, n)],
                                 col8_v.at[pl.ds(0, n)], sem_in),
            ]
            for d in loads:
                d.wait()
            scats = []
            for j in range(kb):
                scats.append(pltpu.async_copy(
                    val8_v.at[pl.ds(j * _SB, _SB)],
                    t8.at[idx_v.at[j]], sem_sc, add=True))
                scats.append(pltpu.async_copy(
                    col8_v.at[pl.ds(j * _SB, _SB)],
                    tc8.at[idx_v.at[j]], sem_sc, add=True))
                scats.append(pltpu.async_copy(
                    ones_v, tcnt.at[idx_v.at[j]], sem_sc, add=True))
            for d in scats:
                d.wait()

        _worker_loop(w, do_chunk)
        plsc.subcore_barrier()

        pltpu.sync_copy(t8.at[pl.ds(r0, _RPS)], out8.at[cid, pl.ds(r0, _RPS)])
        pltpu.sync_copy(tc8.at[pl.ds(r0, _RPS)], outc8.at[cid, pl.ds(r0, _RPS)])
        pltpu.sync_copy(tcnt.at[pl.ds(r0, _RPS)], outcnt.at[cid, pl.ds(r0, _RPS)])

    return k(row2d, val8_2d, plane8, zeros8, zeros1)


_A8 = np.zeros((8, 27), np.float32)
_A1 = np.zeros((1, 27), np.float32)
_B9 = np.zeros((9, 27), np.float32)
_C27 = np.zeros((27, 9), np.float32)
for _i in range(3):
    for _j in range(3):
        for _c in range(3):
            _e = _i * 9 + _j * 3 + _c
            _a = _j * 3 + _c
            if _a < 8:
                _A8[_a, _e] = 1.0
            else:
                _A1[0, _e] = 1.0
            _B9[_i * 3 + _c, _e] = 1.0
            _C27[_e, _i * 3 + _j] = 1.0

_BLK = 1000


def _tc_body(sc_ref, x_ref, t80_ref, t81_ref, c80_ref, c81_ref, cn0_ref,
             cn1_ref, wvd_ref, g16_ref, wvdf_ref, a8_ref, a1_ref, sos_ref,
             sov_ref, sou_ref, bso_ref, wvos_ref, bvos_ref, wvu_ref, k48_ref,
             sout_ref, vout_ref):
    x = x_ref[...]
    vh = jnp.dot(x, wvd_ref[...], preferred_element_type=jnp.float32)
    vnsq = jnp.dot(vh * vh, g16_ref[...], preferred_element_type=jnp.float32)
    vn = jnp.sqrt(vnsq + EPS)
    vdf27 = jnp.dot(x, wvdf_ref[...], preferred_element_type=jnp.float32)
    t8 = t80_ref[...] + t81_ref[...]
    c8 = c80_ref[...] + c81_ref[...]
    cnt = cn0_ref[...] + cn1_ref[...]
    inv = 1.0 / jnp.maximum(cnt, 1.0)
    mean27 = (jnp.dot(t8, a8_ref[...], preferred_element_type=jnp.float32)
              + jnp.dot(c8, a1_ref[...], preferred_element_type=jnp.float32)
              ) * inv
    u = mean27 * vdf27
    s = (jnp.dot(sc_ref[...], sos_ref[...], preferred_element_type=jnp.float32)
         + jnp.dot(vn, sov_ref[...], preferred_element_type=jnp.float32)
         + jnp.dot(u, sou_ref[...], preferred_element_type=jnp.float32)
         + bso_ref[...])
    sil = s * jax.nn.sigmoid(s)
    gate = jnp.dot(sil, wvos_ref[...],
                   preferred_element_type=jnp.float32) + bvos_ref[...]
    g48 = jnp.dot(jax.nn.sigmoid(gate), k48_ref[...],
                  preferred_element_type=jnp.float32)
    vout = jnp.dot(vh, wvu_ref[...], preferred_element_type=jnp.float32) * g48
    sout_ref[...] = sil
    vout_ref[...] = vout


def _tc_node(scalar_rep, x48, t80, t81, c80, c81, cn0, cn1, wvd48, g16,
             wvdf27, a8, a1, sos, sov, sou, bso, wvos_t, bvos, wvu48, k48):
    grid = (N // _BLK,)

    def blk(shape):
        return pl.BlockSpec((_BLK,) + shape[1:], lambda i: (i,) + (0,) * (len(shape) - 1))

    def full(shape):
        return pl.BlockSpec(shape, lambda i: (0,) * len(shape))

    return pl.pallas_call(
        _tc_body,
        grid=grid,
        in_specs=[
            blk((N, S_IN)), blk((N, 48)),
            blk((_N_PAD, 8)), blk((_N_PAD, 8)),
            blk((_N_PAD, 1)), blk((_N_PAD, 1)),
            blk((_N_PAD, 1)), blk((_N_PAD, 1)),
            full((48, 48)), full((48, 16)), full((48, 27)),
            full((8, 27)), full((1, 27)),
            full((S_IN, S_OUT)), full((16, S_OUT)), full((27, S_OUT)),
            full((1, S_OUT)), full((S_OUT, V_OUT)), full((1, V_OUT)),
            full((48, 48)), full((16, 48)),
        ],
        out_specs=[blk((N, S_OUT)), blk((N, 48))],
        out_shape=[
            jax.ShapeDtypeStruct((N, S_OUT), jnp.float32),
            jax.ShapeDtypeStruct((N, 48), jnp.float32),
        ],
    )(scalar_rep, x48, t80, t81, c80, c81, cn0, cn1, wvd48, g16, wvdf27,
      a8, a1, sos, sov, sou, bso, wvos_t, bvos, wvu48, k48)


def kernel(scalar_rep, vector_rep, edge_index, frames,
           W_vd, W_vdf, W_so, b_so, W_vu, W_vos, b_vos):
    row2d = edge_index[0].reshape(E // _SB, _SB)
    planes = [frames[:, j, k] for j in range(3) for k in range(3)]
    val8_2d = _tc_repack([p.reshape(E // 128, 128) for p in planes[:8]],
                         jnp.asarray(_PP)).reshape(E, 8)
    zeros8 = jnp.zeros((_N_PAD, 8), jnp.float32)
    zeros1 = jnp.zeros((_N_PAD,), jnp.float32)
    out8, outc8, outcnt = _sc_segment_sum(
        row2d, val8_2d, planes[8], zeros8, zeros1)

    i3 = jnp.eye(3, dtype=jnp.float32)
    i16 = jnp.eye(16, dtype=jnp.float32)
    wvd48 = jnp.kron(W_vd.T, i3)
    g16 = jnp.kron(i16, jnp.ones((3, 1), jnp.float32))
    wvdf27 = jnp.kron(W_vdf.T, i3) @ jnp.asarray(_B9)
    so_t = W_so.T
    sos = so_t[:S_IN]
    sov = so_t[S_IN:S_IN + HID]
    sou = jnp.asarray(_C27) @ so_t[S_IN + HID:]
    bso = b_so[None, :]
    wvos_t = W_vos.T
    bvos = b_vos[None, :]
    wvu48 = jnp.kron(W_vu.T, i3)
    k48 = jnp.kron(i16, jnp.ones((1, 3), jnp.float32))

    sout, vout48 = _tc_node(
        scalar_rep, vector_rep.reshape(N, 48),
        out8[0], out8[1],
        outc8[0].reshape(_N_PAD, 1), outc8[1].reshape(_N_PAD, 1),
        outcnt[0].reshape(_N_PAD, 1), outcnt[1].reshape(_N_PAD, 1),
        wvd48, g16, wvdf27, jnp.asarray(_A8), jnp.asarray(_A1),
        sos, sov, sou, bso, wvos_t, bvos, wvu48, k48)
    return sout, vout48.reshape(N, V_OUT, 3)


# speedup vs baseline: 5.6488x; 1.0611x over previous
import functools

import numpy as np
import jax
import jax.numpy as jnp
from jax import lax
from jax.experimental import pallas as pl
from jax.experimental.pallas import tpu as pltpu
from jax.experimental.pallas import tpu_sc as plsc

N = 100000
E = 3200000
S_IN = 128
V_IN = 16
S_OUT = 128
V_OUT = 16
HID = 16
EPS = 1e-8

_NW = 32
_SB = 128
_KB = 16
_CHUNK = _SB * _KB
_NCH = 49
_NCH_LAST = 43
_KB_TAIL = 8
_RPS = 6256
_N_PAD = 16 * _RPS
_PSTR = _CHUNK + 8

_SC_PARAMS = pltpu.CompilerParams(use_tc_tiling_on_sc=False)
_MESH = dict(core_axis_name="c", subcore_axis_name="s")


def _worker_loop(w, do_chunk):
    def chunk_body(c, carry):
        do_chunk(w * _NCH + c, _KB)
        return carry

    nch = jnp.where(w < _NW - 1, _NCH, _NCH_LAST)
    lax.fori_loop(0, nch, chunk_body, 0)

    @pl.when(w == _NW - 1)
    def _tail():
        do_chunk((_NW - 1) * _NCH + _NCH_LAST, _KB_TAIL)


_PP = np.zeros((128, 128), np.float32)
for _c in range(8):
    for _k in range(16):
        _PP[_c * 16 + _k, _c + 8 * _k] = 1.0

_RR = 200
_ROWS_P = E // 128
_STEPS = _ROWS_P // _RR


def _tc_repack_body(*refs):
    plane_refs = refs[:8]
    pp_ref, out_ref = refs[8], refs[9]
    pp = pp_ref[...]
    accs = []
    for w in range(8):
        xw = jnp.concatenate(
            [plane_refs[c][:, 16 * w:16 * w + 16] for c in range(8)], axis=1)
        accs.append(jnp.dot(xw, pp, preferred_element_type=jnp.float32))
    stacked = jnp.stack(accs, axis=1)
    out_ref[...] = stacked.reshape(_RR * 8, 128)


def _tc_repack(planes8, pp):
    return pl.pallas_call(
        _tc_repack_body,
        grid=(_STEPS,),
        in_specs=[pl.BlockSpec((_RR, 128), lambda i: (i, 0))
                  for _ in range(8)]
        + [pl.BlockSpec((128, 128), lambda i: (0, 0))],
        out_specs=pl.BlockSpec((_RR * 8, 128), lambda i: (i, 0)),
        out_shape=jax.ShapeDtypeStruct((E // 16, 128), jnp.float32),
    )(*(list(planes8) + [pp]))


def _sc_segment_sum(row2d, val8_2d, plane8, zeros8, zeros1):
    mesh = plsc.VectorSubcoreMesh(**_MESH)

    @functools.partial(
        pl.kernel,
        out_type=[
            jax.ShapeDtypeStruct((2, _N_PAD, 8), jnp.float32),
            jax.ShapeDtypeStruct((2, _N_PAD), jnp.float32),
            jax.ShapeDtypeStruct((2, _N_PAD), jnp.float32),
        ],
        mesh=mesh,
        scratch_types=[
            pltpu.VMEM((2, _KB, _SB), jnp.int32),
            pltpu.VMEM((2, _CHUNK, 8), jnp.float32),
            pltpu.VMEM((2, _CHUNK,), jnp.float32),
            pltpu.VMEM((_SB,), jnp.float32),
            pltpu.VMEM_SHARED((_N_PAD, 8), jnp.float32),
            pltpu.VMEM_SHARED((_N_PAD,), jnp.float32),
            pltpu.VMEM_SHARED((_N_PAD,), jnp.float32),
            pltpu.SemaphoreType.DMA,
            pltpu.SemaphoreType.DMA,
        ],
        compiler_params=_SC_PARAMS,
    )
    def k(row_hbm, val8_hbm, col8_hbm, zeros8_hbm, zeros1_hbm,
          out8, outc8, outcnt, idx_v, val8_v, col8_v, ones_v,
          t8, tc8, tcnt, sem_in, sem_sc):
        cid = lax.axis_index("c")
        sid = lax.axis_index("s")
        w = cid * 16 + sid

        r0 = sid * _RPS
        pltpu.sync_copy(zeros8_hbm.at[pl.ds(r0, _RPS)], t8.at[pl.ds(r0, _RPS)])
        pltpu.sync_copy(zeros1_hbm.at[pl.ds(r0, _RPS)], tc8.at[pl.ds(r0, _RPS)])
        pltpu.sync_copy(zeros1_hbm.at[pl.ds(r0, _RPS)], tcnt.at[pl.ds(r0, _RPS)])

        def fill(i, carry):
            ones_v[pl.ds(i * 16, 16)] = jnp.full((16,), 1.0, jnp.float32)
            return carry

        lax.fori_loop(0, _SB // 16, fill, 0)
        plsc.subcore_barrier()

        def fire_loads(base, p):
            pltpu.async_copy(row_hbm.at[pl.ds(base * _KB, _KB)],
                             idx_v.at[p], sem_in)
            pltpu.async_copy(val8_hbm.at[pl.ds(base * _CHUNK, _CHUNK)],
                             val8_v.at[p], sem_in)
            pltpu.async_copy(col8_hbm.at[pl.ds(base * _CHUNK, _CHUNK)],
                             col8_v.at[p], sem_in)

        def wait_loads(p):
            pltpu.make_async_copy(row_hbm.at[pl.ds(0, _KB)],
                                  idx_v.at[p], sem_in).wait()
            pltpu.make_async_copy(val8_hbm.at[pl.ds(0, _CHUNK)],
                                  val8_v.at[p], sem_in).wait()
            pltpu.make_async_copy(col8_hbm.at[pl.ds(0, _CHUNK)],
                                  col8_v.at[p], sem_in).wait()

        def scatter_chunk(p, kb):
            scats = []
            for j in range(kb):
                scats.append(pltpu.async_copy(
                    val8_v.at[p, pl.ds(j * _SB, _SB)],
                    t8.at[idx_v.at[p, j]], sem_sc, add=True))
                scats.append(pltpu.async_copy(
                    col8_v.at[p, pl.ds(j * _SB, _SB)],
                    tc8.at[idx_v.at[p, j]], sem_sc, add=True))
                scats.append(pltpu.async_copy(
                    ones_v, tcnt.at[idx_v.at[p, j]], sem_sc, add=True))
            for d in scats:
                d.wait()

        nch = jnp.where(w < _NW - 1, _NCH, _NCH_LAST)
        fire_loads(w * _NCH, 0)

        def chunk_body(c, carry):
            p = lax.rem(c, 2)
            wait_loads(p)

            @pl.when(c + 1 < nch)
            def _prefetch():
                fire_loads(w * _NCH + c + 1, 1 - p)

            scatter_chunk(p, _KB)
            return carry

        lax.fori_loop(0, nch, chunk_body, 0)

        @pl.when(w == _NW - 1)
        def _tail():
            base = (_NW - 1) * _NCH + _NCH_LAST
            n = _KB_TAIL * _SB
            pltpu.sync_copy(row_hbm.at[pl.ds(base * _KB, _KB_TAIL)],
                            idx_v.at[0, pl.ds(0, _KB_TAIL)])
            pltpu.sync_copy(val8_hbm.at[pl.ds(base * _CHUNK, n)],
                            val8_v.at[0, pl.ds(0, n)])
            pltpu.sync_copy(col8_hbm.at[pl.ds(base * _CHUNK, n)],
                            col8_v.at[0, pl.ds(0, n)])
            scatter_chunk(0, _KB_TAIL)

        plsc.subcore_barrier()

        pltpu.sync_copy(t8.at[pl.ds(r0, _RPS)], out8.at[cid, pl.ds(r0, _RPS)])
        pltpu.sync_copy(tc8.at[pl.ds(r0, _RPS)], outc8.at[cid, pl.ds(r0, _RPS)])
        pltpu.sync_copy(tcnt.at[pl.ds(r0, _RPS)], outcnt.at[cid, pl.ds(r0, _RPS)])

    return k(row2d, val8_2d, plane8, zeros8, zeros1)


_A8 = np.zeros((8, 27), np.float32)
_A1 = np.zeros((1, 27), np.float32)
_B9 = np.zeros((9, 27), np.float32)
_C27 = np.zeros((27, 9), np.float32)
for _i in range(3):
    for _j in range(3):
        for _c in range(3):
            _e = _i * 9 + _j * 3 + _c
            _a = _j * 3 + _c
            if _a < 8:
                _A8[_a, _e] = 1.0
            else:
                _A1[0, _e] = 1.0
            _B9[_i * 3 + _c, _e] = 1.0
            _C27[_e, _i * 3 + _j] = 1.0

_BLK = 1000


def _tc_body(sc_ref, x_ref, t80_ref, t81_ref, c80_ref, c81_ref, cn0_ref,
             cn1_ref, wvd_ref, g16_ref, wvdf_ref, a8_ref, a1_ref, sos_ref,
             sov_ref, sou_ref, bso_ref, wvos_ref, bvos_ref, wvu_ref, k48_ref,
             sout_ref, vout_ref):
    x = x_ref[...]
    vh = jnp.dot(x, wvd_ref[...], preferred_element_type=jnp.float32)
    vnsq = jnp.dot(vh * vh, g16_ref[...], preferred_element_type=jnp.float32)
    vn = jnp.sqrt(vnsq + EPS)
    vdf27 = jnp.dot(x, wvdf_ref[...], preferred_element_type=jnp.float32)
    t8 = t80_ref[...] + t81_ref[...]
    c8 = c80_ref[...] + c81_ref[...]
    cnt = cn0_ref[...] + cn1_ref[...]
    inv = 1.0 / jnp.maximum(cnt, 1.0)
    mean27 = (jnp.dot(t8, a8_ref[...], preferred_element_type=jnp.float32)
              + jnp.dot(c8, a1_ref[...], preferred_element_type=jnp.float32)
              ) * inv
    u = mean27 * vdf27
    s = (jnp.dot(sc_ref[...], sos_ref[...], preferred_element_type=jnp.float32)
         + jnp.dot(vn, sov_ref[...], preferred_element_type=jnp.float32)
         + jnp.dot(u, sou_ref[...], preferred_element_type=jnp.float32)
         + bso_ref[...])
    sil = s * jax.nn.sigmoid(s)
    gate = jnp.dot(sil, wvos_ref[...],
                   preferred_element_type=jnp.float32) + bvos_ref[...]
    g48 = jnp.dot(jax.nn.sigmoid(gate), k48_ref[...],
                  preferred_element_type=jnp.float32)
    vout = jnp.dot(vh, wvu_ref[...], preferred_element_type=jnp.float32) * g48
    sout_ref[...] = sil
    vout_ref[...] = vout


def _tc_node(scalar_rep, x48, t80, t81, c80, c81, cn0, cn1, wvd48, g16,
             wvdf27, a8, a1, sos, sov, sou, bso, wvos_t, bvos, wvu48, k48):
    grid = (N // _BLK,)

    def blk(shape):
        return pl.BlockSpec((_BLK,) + shape[1:], lambda i: (i,) + (0,) * (len(shape) - 1))

    def full(shape):
        return pl.BlockSpec(shape, lambda i: (0,) * len(shape))

    return pl.pallas_call(
        _tc_body,
        grid=grid,
        in_specs=[
            blk((N, S_IN)), blk((N, 48)),
            blk((_N_PAD, 8)), blk((_N_PAD, 8)),
            blk((_N_PAD, 1)), blk((_N_PAD, 1)),
            blk((_N_PAD, 1)), blk((_N_PAD, 1)),
            full((48, 48)), full((48, 16)), full((48, 27)),
            full((8, 27)), full((1, 27)),
            full((S_IN, S_OUT)), full((16, S_OUT)), full((27, S_OUT)),
            full((1, S_OUT)), full((S_OUT, V_OUT)), full((1, V_OUT)),
            full((48, 48)), full((16, 48)),
        ],
        out_specs=[blk((N, S_OUT)), blk((N, 48))],
        out_shape=[
            jax.ShapeDtypeStruct((N, S_OUT), jnp.float32),
            jax.ShapeDtypeStruct((N, 48), jnp.float32),
        ],
    )(scalar_rep, x48, t80, t81, c80, c81, cn0, cn1, wvd48, g16, wvdf27,
      a8, a1, sos, sov, sou, bso, wvos_t, bvos, wvu48, k48)


def kernel(scalar_rep, vector_rep, edge_index, frames,
           W_vd, W_vdf, W_so, b_so, W_vu, W_vos, b_vos):
    row2d = edge_index[0].reshape(E // _SB, _SB)
    planes = [frames[:, j, k] for j in range(3) for k in range(3)]
    val8_2d = _tc_repack([p.reshape(E // 128, 128) for p in planes[:8]],
                         jnp.asarray(_PP)).reshape(E, 8)
    zeros8 = jnp.zeros((_N_PAD, 8), jnp.float32)
    zeros1 = jnp.zeros((_N_PAD,), jnp.float32)
    out8, outc8, outcnt = _sc_segment_sum(
        row2d, val8_2d, planes[8], zeros8, zeros1)

    i3 = jnp.eye(3, dtype=jnp.float32)
    i16 = jnp.eye(16, dtype=jnp.float32)
    wvd48 = jnp.kron(W_vd.T, i3)
    g16 = jnp.kron(i16, jnp.ones((3, 1), jnp.float32))
    wvdf27 = jnp.kron(W_vdf.T, i3) @ jnp.asarray(_B9)
    so_t = W_so.T
    sos = so_t[:S_IN]
    sov = so_t[S_IN:S_IN + HID]
    sou = jnp.asarray(_C27) @ so_t[S_IN + HID:]
    bso = b_so[None, :]
    wvos_t = W_vos.T
    bvos = b_vos[None, :]
    wvu48 = jnp.kron(W_vu.T, i3)
    k48 = jnp.kron(i16, jnp.ones((1, 3), jnp.float32))

    sout, vout48 = _tc_node(
        scalar_rep, vector_rep.reshape(N, 48),
        out8[0], out8[1],
        outc8[0].reshape(_N_PAD, 1), outc8[1].reshape(_N_PAD, 1),
        outcnt[0].reshape(_N_PAD, 1), outcnt[1].reshape(_N_PAD, 1),
        wvd48, g16, wvdf27, jnp.asarray(_A8), jnp.asarray(_A1),
        sos, sov, sou, bso, wvos_t, bvos, wvu48, k48)
    return sout, vout48.reshape(N, V_OUT, 3)
